# baseline (speedup 1.0000x reference)
import functools
import math

import jax
import jax.numpy as jnp
from jax import lax
from jax.experimental import pallas as pl
from jax.experimental.pallas import tpu as pltpu

_D = 768
_E = 64
_F = 64
_K = 8
_N = 2048
_BM = 256
_NC = 8
_CW = _E * _F // _NC


def _gate_from_logits(logits):
    sel = jax.nn.sigmoid(logits)

    def _bcast(v):
        return jnp.broadcast_to(v, (_BM, _E))

    def step(_, carry):
        t, cnt, cnt_gt = carry
        active = cnt < _K
        masked = jnp.where(sel < t, sel, -jnp.inf)
        m = _bcast(jnp.max(masked, axis=1, keepdims=True))
        n_eq = _bcast(jnp.sum((sel == m).astype(jnp.float32), axis=1,
                              keepdims=True))
        t = jnp.where(active, m, t)
        cnt_gt = jnp.where(active, cnt, cnt_gt)
        cnt = jnp.where(active, cnt + n_eq, cnt)
        return t, cnt, cnt_gt

    zeros = sel * 0.0
    t, _, cnt_gt = lax.fori_loop(0, _K, step, (zeros + jnp.inf, zeros, zeros))

    eq = (sel == t).astype(jnp.float32)
    row = lax.broadcasted_iota(jnp.int32, (_E, _E), 0)
    col = lax.broadcasted_iota(jnp.int32, (_E, _E), 1)
    tril = (row < col).astype(jnp.float32)
    excl = lax.dot_general(eq, tril, (((1,), (0,)), ((), ())),
                           preferred_element_type=jnp.float32)
    keep = (sel > t) | ((eq > 0) & (excl < (_K - cnt_gt)))
    return jnp.where(keep, sel, 0.0)


def _moe_body(x_ref, esel_ref, kflat_ref, vflat_ref, out_ref,
              gate_x_ref, xb_ref):
    j = pl.program_id(1)

    @pl.when(j == 0)
    def _prep():
        x = x_ref[...]
        logits = lax.dot_general(
            x, esel_ref[...], (((1,), (1,)), ((), ())),
            preferred_element_type=jnp.float32)
        gate = _gate_from_logits(logits)
        erow = lax.broadcasted_iota(jnp.int32, (_E, _E * _F), 0)
        ecol = lax.broadcasted_iota(jnp.int32, (_E, _E * _F), 1)
        expand = (ecol // _F == erow).astype(jnp.float32)
        gate_x_ref[...] = lax.dot_general(
            gate, expand, (((1,), (0,)), ((), ())),
            preferred_element_type=jnp.float32)
        xb_ref[...] = x.astype(jnp.bfloat16)

    h = lax.dot_general(xb_ref[...], kflat_ref[...], (((1,), (0,)), ((), ())),
                        preferred_element_type=jnp.float32)
    h = jnp.maximum(h, 0.0) * gate_x_ref[:, pl.ds(j * _CW, _CW)]
    acc = lax.dot_general(h.astype(jnp.bfloat16), vflat_ref[...],
                          (((1,), (0,)), ((), ())),
                          preferred_element_type=jnp.float32)

    @pl.when(j == 0)
    def _init():
        out_ref[...] = acc

    @pl.when(j > 0)
    def _acc():
        out_ref[...] = out_ref[...] + acc


@jax.jit
def kernel(input, expert_sel, keys, values):
    kflat = jnp.transpose(keys, (1, 0, 2)).reshape(_D, _E * _F)
    kflat = kflat.astype(jnp.bfloat16)
    vflat = values.reshape(_E * _F, _D).astype(jnp.bfloat16)
    out = pl.pallas_call(
        _moe_body,
        grid=(_N // _BM, _NC),
        in_specs=[
            pl.BlockSpec((_BM, _D), lambda i, j: (i, 0)),
            pl.BlockSpec((_E, _D), lambda i, j: (0, 0)),
            pl.BlockSpec((_D, _CW), lambda i, j: (0, j)),
            pl.BlockSpec((_CW, _D), lambda i, j: (j, 0)),
        ],
        out_specs=pl.BlockSpec((_BM, _D), lambda i, j: (i, 0)),
        out_shape=jax.ShapeDtypeStruct((_N, _D), jnp.float32),
        scratch_shapes=[
            pltpu.VMEM((_BM, _E * _F), jnp.float32),
            pltpu.VMEM((_BM, _D), jnp.bfloat16),
        ],
        compiler_params=pltpu.CompilerParams(
            dimension_semantics=("parallel", "arbitrary")),
    )(input, expert_sel, kflat, vflat)
    return out


# speedup vs baseline: 1.8758x; 1.8758x over previous
import functools
import math

import jax
import jax.numpy as jnp
from jax import lax
from jax.experimental import pallas as pl
from jax.experimental.pallas import tpu as pltpu

_D = 768
_E = 64
_F = 64
_K = 8
_N = 2048
_BM = 512


def _moe_body(x_ref, esel_ref, kflat_ref, vflat_ref, out_ref):
    x = x_ref[...]
    logits = lax.dot_general(
        x, esel_ref[...], (((1,), (1,)), ((), ())),
        preferred_element_type=jnp.float32)
    sel = jax.nn.sigmoid(logits)

    def _bcast(v):
        return jnp.broadcast_to(v, (_BM, _E))

    def step(_, carry):
        t, cnt, cnt_gt = carry
        active = cnt < _K
        masked = jnp.where(sel < t, sel, -jnp.inf)
        m = _bcast(jnp.max(masked, axis=1, keepdims=True))
        n_eq = _bcast(jnp.sum((sel == m).astype(jnp.float32), axis=1,
                              keepdims=True))
        t = jnp.where(active, m, t)
        cnt_gt = jnp.where(active, cnt, cnt_gt)
        cnt = jnp.where(active, cnt + n_eq, cnt)
        return t, cnt, cnt_gt

    zeros = sel * 0.0
    t, _, cnt_gt = lax.fori_loop(0, _K, step, (zeros + jnp.inf, zeros, zeros))

    eq = (sel == t).astype(jnp.float32)
    row = lax.broadcasted_iota(jnp.int32, (_E, _E), 0)
    col = lax.broadcasted_iota(jnp.int32, (_E, _E), 1)
    tril = (row < col).astype(jnp.float32)
    excl = lax.dot_general(eq, tril, (((1,), (0,)), ((), ())),
                           preferred_element_type=jnp.float32)
    keep = (sel > t) | ((eq > 0) & (excl < (_K - cnt_gt)))
    gate = jnp.where(keep, sel, 0.0)

    erow = lax.broadcasted_iota(jnp.int32, (_E, _E * _F), 0)
    ecol = lax.broadcasted_iota(jnp.int32, (_E, _E * _F), 1)
    expand = (ecol // _F == erow).astype(jnp.float32)
    gate_x = lax.dot_general(gate, expand, (((1,), (0,)), ((), ())),
                             preferred_element_type=jnp.float32)

    h = lax.dot_general(x, kflat_ref[...], (((1,), (0,)), ((), ())),
                        preferred_element_type=jnp.float32)
    h = jnp.maximum(h, 0.0) * gate_x
    out_ref[...] = lax.dot_general(
        h, vflat_ref[...], (((1,), (0,)), ((), ())),
        preferred_element_type=jnp.float32)


@jax.jit
def kernel(input, expert_sel, keys, values):
    kflat = jnp.transpose(keys, (1, 0, 2)).reshape(_D, _E * _F)
    vflat = values.reshape(_E * _F, _D)
    out = pl.pallas_call(
        _moe_body,
        grid=(_N // _BM,),
        in_specs=[
            pl.BlockSpec((_BM, _D), lambda i: (i, 0)),
            pl.BlockSpec((_E, _D), lambda i: (0, 0)),
            pl.BlockSpec((_D, _E * _F), lambda i: (0, 0)),
            pl.BlockSpec((_E * _F, _D), lambda i: (0, 0)),
        ],
        out_specs=pl.BlockSpec((_BM, _D), lambda i: (i, 0)),
        out_shape=jax.ShapeDtypeStruct((_N, _D), jnp.float32),
    )(input, expert_sel, kflat, vflat)
    return out
